# baseline (speedup 1.0000x reference)
import functools

import jax
import jax.numpy as jnp
from jax import lax
from jax.experimental import pallas as pl
from jax.experimental.pallas import tpu as pltpu
from jax.experimental.pallas import tpu_sc as plsc

N_CODES = 8192
C_DIM = 256
T_TILE = 256
N_BLK = 1024


def _dist_argmin_body(z2_ref, e_ref, en_ref, idx_ref, val_ref):
    z2 = z2_ref[...]
    z_norm = 0.25 * jnp.sum(z2 * z2, axis=1, keepdims=True)
    iota = jax.lax.broadcasted_iota(jnp.int32, (T_TILE, N_BLK), 1)

    def body(j, carry):
        run_val, run_idx = carry
        e_blk = e_ref[pl.ds(j * N_BLK, N_BLK), :]
        mm2 = jax.lax.dot_general(
            z2.astype(jnp.bfloat16), e_blk.astype(jnp.bfloat16),
            (((1,), (1,)), ((), ())), preferred_element_type=jnp.float32)
        e_norm = en_ref[0, 0, pl.ds(j * N_BLK, N_BLK)]
        dist = (z_norm + e_norm[None, :]) - mm2
        bmin = jnp.min(dist, axis=1)
        masked = jnp.where(dist == bmin[:, None], iota, jnp.int32(N_BLK))
        bidx = jnp.min(masked, axis=1) + j * N_BLK
        upd = bmin < run_val
        return (jnp.where(upd, bmin, run_val), jnp.where(upd, bidx, run_idx))

    init = (jnp.full((T_TILE,), jnp.inf, jnp.float32),
            jnp.zeros((T_TILE,), jnp.int32))
    run_val, run_idx = jax.lax.fori_loop(0, N_CODES // N_BLK, body, init)
    idx_ref[0, 0, :] = run_idx
    val_ref[0, 0, :] = run_val


def _dist_argmin(z_flat, embedding_weight):
    t = z_flat.shape[0]
    grid = (t // T_TILE,)
    e_norm = jnp.sum(embedding_weight ** 2, axis=1).reshape(1, 1, N_CODES)
    idx3, val3 = pl.pallas_call(
        _dist_argmin_body,
        grid=grid,
        in_specs=[
            pl.BlockSpec((T_TILE, C_DIM), lambda i: (i, 0)),
            pl.BlockSpec((N_CODES, C_DIM), lambda i: (0, 0)),
            pl.BlockSpec((1, 1, N_CODES), lambda i: (0, 0, 0)),
        ],
        out_specs=[
            pl.BlockSpec((1, 1, T_TILE), lambda i: (i, 0, 0)),
            pl.BlockSpec((1, 1, T_TILE), lambda i: (i, 0, 0)),
        ],
        out_shape=[
            jax.ShapeDtypeStruct((grid[0], 1, T_TILE), jnp.int32),
            jax.ShapeDtypeStruct((grid[0], 1, T_TILE), jnp.float32),
        ],
    )(2.0 * z_flat, embedding_weight, e_norm)
    return idx3.reshape(-1), val3.reshape(-1)


_SC_CHUNK = 256


def _make_sc_gather(n_tokens):
    info = plsc.get_sparse_core_info()
    nc, ns = info.num_cores, info.num_subcores
    nw = nc * ns
    b_per_w = n_tokens // nw
    n_chunks = b_per_w // _SC_CHUNK
    mesh = plsc.VectorSubcoreMesh(core_axis_name="c", subcore_axis_name="s")

    @functools.partial(
        pl.kernel, mesh=mesh,
        out_type=jax.ShapeDtypeStruct((n_tokens, C_DIM), jnp.float32),
        scratch_types=[
            pltpu.VMEM((_SC_CHUNK,), jnp.int32),
            pltpu.VMEM((_SC_CHUNK, C_DIM), jnp.float32),
            pltpu.SemaphoreType.DMA,
        ],
    )
    def gather_k(table_hbm, idx_hbm, out_hbm, idx_v, rows_v, sem):
        wid = lax.axis_index("s") * nc + lax.axis_index("c")
        base = wid * b_per_w
        for ci in range(n_chunks):
            off = base + ci * _SC_CHUNK
            pltpu.sync_copy(idx_hbm.at[pl.ds(off, _SC_CHUNK)], idx_v)
            pltpu.async_copy(table_hbm.at[idx_v], rows_v, sem).wait()
            pltpu.sync_copy(rows_v, out_hbm.at[pl.ds(off, _SC_CHUNK)])

    return gather_k


def kernel(z, embedding_weight):
    b, c, d_, h, w = z.shape
    z_flat = jnp.transpose(z, (0, 2, 3, 4, 1)).reshape(-1, c)
    min_encoding_indices, min_vals = _dist_argmin(z_flat, embedding_weight)
    z_q_flat = _make_sc_gather(z_flat.shape[0])(
        embedding_weight, min_encoding_indices)
    z_q = jnp.transpose(z_q_flat.reshape(b, d_, h, w, c), (0, 4, 1, 2, 3))
    loss = 1.25 * jnp.sum(min_vals) / jnp.float32(z.size)
    indices = min_encoding_indices.reshape(b, d_, h, w)
    return (z_q, loss, indices)


# speedup vs baseline: 1.0619x; 1.0619x over previous
import functools

import jax
import jax.numpy as jnp
from jax import lax
from jax.experimental import pallas as pl
from jax.experimental.pallas import tpu as pltpu
from jax.experimental.pallas import tpu_sc as plsc

N_CODES = 8192
C_DIM = 256
T_TILE = 256
N_BLK = 1024


def _dist_argmin_body(z_ref, e_ref, idx_ref, val_ref):
    z = z_ref[...]
    z_norm = jnp.sum(z * z, axis=1, keepdims=True)

    def body(j, carry):
        run_val, run_idx = carry
        e_blk = e_ref[pl.ds(j * N_BLK, N_BLK), :]
        mm = jax.lax.dot_general(z, e_blk, (((1,), (1,)), ((), ())))
        e_norm = jnp.sum(e_blk * e_blk, axis=1)
        dist = (z_norm + e_norm[None, :]) - 2.0 * mm
        bmin = jnp.min(dist, axis=1)
        iota = jax.lax.broadcasted_iota(jnp.int32, dist.shape, 1)
        masked = jnp.where(dist == bmin[:, None], iota, jnp.int32(N_BLK))
        bidx = jnp.min(masked, axis=1) + j * N_BLK
        upd = bmin < run_val
        return (jnp.where(upd, bmin, run_val), jnp.where(upd, bidx, run_idx))

    init = (jnp.full((T_TILE,), jnp.inf, jnp.float32),
            jnp.zeros((T_TILE,), jnp.int32))
    run_val, run_idx = jax.lax.fori_loop(0, N_CODES // N_BLK, body, init)
    idx_ref[0, 0, :] = run_idx
    val_ref[0, 0, :] = run_val


def _dist_argmin(z_flat, embedding_weight):
    t = z_flat.shape[0]
    grid = (t // T_TILE,)
    idx3, val3 = pl.pallas_call(
        _dist_argmin_body,
        grid=grid,
        in_specs=[
            pl.BlockSpec((T_TILE, C_DIM), lambda i: (i, 0)),
            pl.BlockSpec((N_CODES, C_DIM), lambda i: (0, 0)),
        ],
        out_specs=[
            pl.BlockSpec((1, 1, T_TILE), lambda i: (i, 0, 0)),
            pl.BlockSpec((1, 1, T_TILE), lambda i: (i, 0, 0)),
        ],
        out_shape=[
            jax.ShapeDtypeStruct((grid[0], 1, T_TILE), jnp.int32),
            jax.ShapeDtypeStruct((grid[0], 1, T_TILE), jnp.float32),
        ],
    )(z_flat, embedding_weight)
    return idx3.reshape(-1), val3.reshape(-1)


_SC_CHUNK = 256


def _make_sc_gather(n_tokens):
    info = plsc.get_sparse_core_info()
    nc, ns = info.num_cores, info.num_subcores
    nw = nc * ns
    b_per_w = n_tokens // nw
    n_chunks = b_per_w // _SC_CHUNK
    mesh = plsc.VectorSubcoreMesh(core_axis_name="c", subcore_axis_name="s")

    @functools.partial(
        pl.kernel, mesh=mesh,
        out_type=jax.ShapeDtypeStruct((n_tokens, C_DIM), jnp.float32),
        scratch_types=[
            pltpu.VMEM((_SC_CHUNK,), jnp.int32),
            pltpu.VMEM((_SC_CHUNK, C_DIM), jnp.float32),
            pltpu.SemaphoreType.DMA,
        ],
    )
    def gather_k(table_hbm, idx_hbm, out_hbm, idx_v, rows_v, sem):
        wid = lax.axis_index("s") * nc + lax.axis_index("c")
        base = wid * b_per_w
        for ci in range(n_chunks):
            off = base + ci * _SC_CHUNK
            pltpu.sync_copy(idx_hbm.at[pl.ds(off, _SC_CHUNK)], idx_v)
            pltpu.async_copy(table_hbm.at[idx_v], rows_v, sem).wait()
            pltpu.sync_copy(rows_v, out_hbm.at[pl.ds(off, _SC_CHUNK)])

    return gather_k


def kernel(z, embedding_weight):
    b, c, d_, h, w = z.shape
    z_flat = jnp.transpose(z, (0, 2, 3, 4, 1)).reshape(-1, c)
    min_encoding_indices, min_vals = _dist_argmin(z_flat, embedding_weight)
    z_q_flat = _make_sc_gather(z_flat.shape[0])(
        embedding_weight, min_encoding_indices)
    z_q = jnp.transpose(z_q_flat.reshape(b, d_, h, w, c), (0, 4, 1, 2, 3))
    loss = 1.25 * jnp.sum(min_vals) / jnp.float32(z.size)
    indices = min_encoding_indices.reshape(b, d_, h, w)
    return (z_q, loss, indices)


# speedup vs baseline: 1.3135x; 1.2369x over previous
import functools

import jax
import jax.numpy as jnp
from jax import lax
from jax.experimental import pallas as pl
from jax.experimental.pallas import tpu as pltpu
from jax.experimental.pallas import tpu_sc as plsc

N_CODES = 8192
C_DIM = 256
T_TILE = 512
N_BLK = 1024


def _dist_argmin_body(z_ref, e_ref, idx_ref, val_ref):
    z = z_ref[...]
    z_norm = jnp.sum(z * z, axis=1, keepdims=True)

    def body(j, carry):
        run_val, run_idx = carry
        e_blk = e_ref[pl.ds(j * N_BLK, N_BLK), :]
        mm = jax.lax.dot_general(z, e_blk, (((1,), (1,)), ((), ())))
        e_norm = jnp.sum(e_blk * e_blk, axis=1)
        dist = (z_norm + e_norm[None, :]) - 2.0 * mm
        bmin = jnp.min(dist, axis=1)
        iota = jax.lax.broadcasted_iota(jnp.int32, dist.shape, 1)
        masked = jnp.where(dist == bmin[:, None], iota, jnp.int32(N_BLK))
        bidx = jnp.min(masked, axis=1) + j * N_BLK
        upd = bmin < run_val
        return (jnp.where(upd, bmin, run_val), jnp.where(upd, bidx, run_idx))

    init = (jnp.full((T_TILE,), jnp.inf, jnp.float32),
            jnp.zeros((T_TILE,), jnp.int32))
    run_val, run_idx = jax.lax.fori_loop(0, N_CODES // N_BLK, body, init)
    idx_ref[0, 0, :] = run_idx
    val_ref[0, 0, :] = run_val


def _dist_argmin(z_flat, embedding_weight):
    t = z_flat.shape[0]
    grid = (t // T_TILE,)
    idx3, val3 = pl.pallas_call(
        _dist_argmin_body,
        grid=grid,
        in_specs=[
            pl.BlockSpec((T_TILE, C_DIM), lambda i: (i, 0)),
            pl.BlockSpec((N_CODES, C_DIM), lambda i: (0, 0)),
        ],
        out_specs=[
            pl.BlockSpec((1, 1, T_TILE), lambda i: (i, 0, 0)),
            pl.BlockSpec((1, 1, T_TILE), lambda i: (i, 0, 0)),
        ],
        out_shape=[
            jax.ShapeDtypeStruct((grid[0], 1, T_TILE), jnp.int32),
            jax.ShapeDtypeStruct((grid[0], 1, T_TILE), jnp.float32),
        ],
    )(z_flat, embedding_weight)
    return idx3.reshape(-1), val3.reshape(-1)


_SC_CHUNK = 256


def _make_sc_gather(n_tokens):
    info = plsc.get_sparse_core_info()
    nc, ns = info.num_cores, info.num_subcores
    nw = nc * ns
    b_per_w = n_tokens // nw
    n_chunks = b_per_w // _SC_CHUNK
    mesh = plsc.VectorSubcoreMesh(core_axis_name="c", subcore_axis_name="s")

    @functools.partial(
        pl.kernel, mesh=mesh,
        out_type=jax.ShapeDtypeStruct((n_tokens, C_DIM), jnp.float32),
        scratch_types=[
            pltpu.VMEM((_SC_CHUNK,), jnp.int32),
            pltpu.VMEM((_SC_CHUNK, C_DIM), jnp.float32),
            pltpu.SemaphoreType.DMA,
        ],
    )
    def gather_k(table_hbm, idx_hbm, out_hbm, idx_v, rows_v, sem):
        wid = lax.axis_index("s") * nc + lax.axis_index("c")
        base = wid * b_per_w
        for ci in range(n_chunks):
            off = base + ci * _SC_CHUNK
            pltpu.sync_copy(idx_hbm.at[pl.ds(off, _SC_CHUNK)], idx_v)
            pltpu.async_copy(table_hbm.at[idx_v], rows_v, sem).wait()
            pltpu.sync_copy(rows_v, out_hbm.at[pl.ds(off, _SC_CHUNK)])

    return gather_k


def kernel(z, embedding_weight):
    b, c, d_, h, w = z.shape
    z_flat = jnp.transpose(z, (0, 2, 3, 4, 1)).reshape(-1, c)
    min_encoding_indices, min_vals = _dist_argmin(z_flat, embedding_weight)
    z_q_flat = _make_sc_gather(z_flat.shape[0])(
        embedding_weight, min_encoding_indices)
    z_q = jnp.transpose(z_q_flat.reshape(b, d_, h, w, c), (0, 4, 1, 2, 3))
    loss = 1.25 * jnp.sum(min_vals) / jnp.float32(z.size)
    indices = min_encoding_indices.reshape(b, d_, h, w)
    return (z_q, loss, indices)


# speedup vs baseline: 1.6004x; 1.2184x over previous
import functools

import jax
import jax.numpy as jnp
from jax import lax
from jax.experimental import pallas as pl
from jax.experimental.pallas import tpu as pltpu
from jax.experimental.pallas import tpu_sc as plsc

N_CODES = 8192
C_DIM = 256
T_TILE = 1024
N_BLK = 1024


def _dist_argmin_body(z_ref, e_ref, idx_ref, val_ref):
    z = z_ref[...]
    z_norm = jnp.sum(z * z, axis=1, keepdims=True)

    def body(j, carry):
        run_val, run_idx = carry
        e_blk = e_ref[pl.ds(j * N_BLK, N_BLK), :]
        mm = jax.lax.dot_general(z, e_blk, (((1,), (1,)), ((), ())))
        e_norm = jnp.sum(e_blk * e_blk, axis=1)
        dist = (z_norm + e_norm[None, :]) - 2.0 * mm
        bmin = jnp.min(dist, axis=1)
        iota = jax.lax.broadcasted_iota(jnp.int32, dist.shape, 1)
        masked = jnp.where(dist == bmin[:, None], iota, jnp.int32(N_BLK))
        bidx = jnp.min(masked, axis=1) + j * N_BLK
        upd = bmin < run_val
        return (jnp.where(upd, bmin, run_val), jnp.where(upd, bidx, run_idx))

    init = (jnp.full((T_TILE,), jnp.inf, jnp.float32),
            jnp.zeros((T_TILE,), jnp.int32))
    run_val, run_idx = jax.lax.fori_loop(0, N_CODES // N_BLK, body, init)
    idx_ref[0, 0, :] = run_idx
    val_ref[0, 0, :] = run_val


def _dist_argmin(z_flat, embedding_weight):
    t = z_flat.shape[0]
    grid = (t // T_TILE,)
    idx3, val3 = pl.pallas_call(
        _dist_argmin_body,
        grid=grid,
        in_specs=[
            pl.BlockSpec((T_TILE, C_DIM), lambda i: (i, 0)),
            pl.BlockSpec((N_CODES, C_DIM), lambda i: (0, 0)),
        ],
        out_specs=[
            pl.BlockSpec((1, 1, T_TILE), lambda i: (i, 0, 0)),
            pl.BlockSpec((1, 1, T_TILE), lambda i: (i, 0, 0)),
        ],
        out_shape=[
            jax.ShapeDtypeStruct((grid[0], 1, T_TILE), jnp.int32),
            jax.ShapeDtypeStruct((grid[0], 1, T_TILE), jnp.float32),
        ],
    )(z_flat, embedding_weight)
    return idx3.reshape(-1), val3.reshape(-1)


_SC_CHUNK = 256


def _make_sc_gather(n_tokens):
    info = plsc.get_sparse_core_info()
    nc, ns = info.num_cores, info.num_subcores
    nw = nc * ns
    b_per_w = n_tokens // nw
    n_chunks = b_per_w // _SC_CHUNK
    mesh = plsc.VectorSubcoreMesh(core_axis_name="c", subcore_axis_name="s")

    @functools.partial(
        pl.kernel, mesh=mesh,
        out_type=jax.ShapeDtypeStruct((n_tokens, C_DIM), jnp.float32),
        scratch_types=[
            pltpu.VMEM((_SC_CHUNK,), jnp.int32),
            pltpu.VMEM((_SC_CHUNK, C_DIM), jnp.float32),
            pltpu.SemaphoreType.DMA,
        ],
    )
    def gather_k(table_hbm, idx_hbm, out_hbm, idx_v, rows_v, sem):
        wid = lax.axis_index("s") * nc + lax.axis_index("c")
        base = wid * b_per_w
        for ci in range(n_chunks):
            off = base + ci * _SC_CHUNK
            pltpu.sync_copy(idx_hbm.at[pl.ds(off, _SC_CHUNK)], idx_v)
            pltpu.async_copy(table_hbm.at[idx_v], rows_v, sem).wait()
            pltpu.sync_copy(rows_v, out_hbm.at[pl.ds(off, _SC_CHUNK)])

    return gather_k


def kernel(z, embedding_weight):
    b, c, d_, h, w = z.shape
    z_flat = jnp.transpose(z, (0, 2, 3, 4, 1)).reshape(-1, c)
    min_encoding_indices, min_vals = _dist_argmin(z_flat, embedding_weight)
    z_q_flat = _make_sc_gather(z_flat.shape[0])(
        embedding_weight, min_encoding_indices)
    z_q = jnp.transpose(z_q_flat.reshape(b, d_, h, w, c), (0, 4, 1, 2, 3))
    loss = 1.25 * jnp.sum(min_vals) / jnp.float32(z.size)
    indices = min_encoding_indices.reshape(b, d_, h, w)
    return (z_q, loss, indices)


# speedup vs baseline: 1.7414x; 1.0881x over previous
import functools

import jax
import jax.numpy as jnp
from jax import lax
from jax.experimental import pallas as pl
from jax.experimental.pallas import tpu as pltpu
from jax.experimental.pallas import tpu_sc as plsc

N_CODES = 8192
C_DIM = 256
T_TILE = 2048
N_BLK = 1024


def _dist_argmin_body(z_ref, e_ref, idx_ref, val_ref):
    z = z_ref[...]
    z_norm = jnp.sum(z * z, axis=1, keepdims=True)

    def body(j, carry):
        run_val, run_idx = carry
        e_blk = e_ref[pl.ds(j * N_BLK, N_BLK), :]
        mm = jax.lax.dot_general(z, e_blk, (((1,), (1,)), ((), ())))
        e_norm = jnp.sum(e_blk * e_blk, axis=1)
        dist = (z_norm + e_norm[None, :]) - 2.0 * mm
        bmin = jnp.min(dist, axis=1)
        iota = jax.lax.broadcasted_iota(jnp.int32, dist.shape, 1)
        masked = jnp.where(dist == bmin[:, None], iota, jnp.int32(N_BLK))
        bidx = jnp.min(masked, axis=1) + j * N_BLK
        upd = bmin < run_val
        return (jnp.where(upd, bmin, run_val), jnp.where(upd, bidx, run_idx))

    init = (jnp.full((T_TILE,), jnp.inf, jnp.float32),
            jnp.zeros((T_TILE,), jnp.int32))
    run_val, run_idx = jax.lax.fori_loop(0, N_CODES // N_BLK, body, init)
    idx_ref[0, 0, :] = run_idx
    val_ref[0, 0, :] = run_val


def _dist_argmin(z_flat, embedding_weight):
    t = z_flat.shape[0]
    grid = (t // T_TILE,)
    idx3, val3 = pl.pallas_call(
        _dist_argmin_body,
        grid=grid,
        in_specs=[
            pl.BlockSpec((T_TILE, C_DIM), lambda i: (i, 0)),
            pl.BlockSpec((N_CODES, C_DIM), lambda i: (0, 0)),
        ],
        out_specs=[
            pl.BlockSpec((1, 1, T_TILE), lambda i: (i, 0, 0)),
            pl.BlockSpec((1, 1, T_TILE), lambda i: (i, 0, 0)),
        ],
        out_shape=[
            jax.ShapeDtypeStruct((grid[0], 1, T_TILE), jnp.int32),
            jax.ShapeDtypeStruct((grid[0], 1, T_TILE), jnp.float32),
        ],
    )(z_flat, embedding_weight)
    return idx3.reshape(-1), val3.reshape(-1)


_SC_CHUNK = 256


def _make_sc_gather(n_tokens):
    info = plsc.get_sparse_core_info()
    nc, ns = info.num_cores, info.num_subcores
    nw = nc * ns
    b_per_w = n_tokens // nw
    n_chunks = b_per_w // _SC_CHUNK
    mesh = plsc.VectorSubcoreMesh(core_axis_name="c", subcore_axis_name="s")

    @functools.partial(
        pl.kernel, mesh=mesh,
        out_type=jax.ShapeDtypeStruct((n_tokens, C_DIM), jnp.float32),
        scratch_types=[
            pltpu.VMEM((_SC_CHUNK,), jnp.int32),
            pltpu.VMEM((_SC_CHUNK, C_DIM), jnp.float32),
            pltpu.SemaphoreType.DMA,
        ],
    )
    def gather_k(table_hbm, idx_hbm, out_hbm, idx_v, rows_v, sem):
        wid = lax.axis_index("s") * nc + lax.axis_index("c")
        base = wid * b_per_w
        for ci in range(n_chunks):
            off = base + ci * _SC_CHUNK
            pltpu.sync_copy(idx_hbm.at[pl.ds(off, _SC_CHUNK)], idx_v)
            pltpu.async_copy(table_hbm.at[idx_v], rows_v, sem).wait()
            pltpu.sync_copy(rows_v, out_hbm.at[pl.ds(off, _SC_CHUNK)])

    return gather_k


def kernel(z, embedding_weight):
    b, c, d_, h, w = z.shape
    z_flat = jnp.transpose(z, (0, 2, 3, 4, 1)).reshape(-1, c)
    min_encoding_indices, min_vals = _dist_argmin(z_flat, embedding_weight)
    z_q_flat = _make_sc_gather(z_flat.shape[0])(
        embedding_weight, min_encoding_indices)
    z_q = jnp.transpose(z_q_flat.reshape(b, d_, h, w, c), (0, 4, 1, 2, 3))
    loss = 1.25 * jnp.sum(min_vals) / jnp.float32(z.size)
    indices = min_encoding_indices.reshape(b, d_, h, w)
    return (z_q, loss, indices)


# speedup vs baseline: 1.8121x; 1.0406x over previous
import functools

import jax
import jax.numpy as jnp
from jax import lax
from jax.experimental import pallas as pl
from jax.experimental.pallas import tpu as pltpu
from jax.experimental.pallas import tpu_sc as plsc

N_CODES = 8192
C_DIM = 256
T_TILE = 4096
N_BLK = 1024


def _dist_argmin_body(z_ref, e_ref, idx_ref, val_ref):
    z = z_ref[...]
    z_norm = jnp.sum(z * z, axis=1, keepdims=True)

    def body(j, carry):
        run_val, run_idx = carry
        e_blk = e_ref[pl.ds(j * N_BLK, N_BLK), :]
        mm = jax.lax.dot_general(z, e_blk, (((1,), (1,)), ((), ())))
        e_norm = jnp.sum(e_blk * e_blk, axis=1)
        dist = (z_norm + e_norm[None, :]) - 2.0 * mm
        bmin = jnp.min(dist, axis=1)
        iota = jax.lax.broadcasted_iota(jnp.int32, dist.shape, 1)
        masked = jnp.where(dist == bmin[:, None], iota, jnp.int32(N_BLK))
        bidx = jnp.min(masked, axis=1) + j * N_BLK
        upd = bmin < run_val
        return (jnp.where(upd, bmin, run_val), jnp.where(upd, bidx, run_idx))

    init = (jnp.full((T_TILE,), jnp.inf, jnp.float32),
            jnp.zeros((T_TILE,), jnp.int32))
    run_val, run_idx = jax.lax.fori_loop(0, N_CODES // N_BLK, body, init)
    idx_ref[0, 0, :] = run_idx
    val_ref[0, 0, :] = run_val


def _dist_argmin(z_flat, embedding_weight):
    t = z_flat.shape[0]
    grid = (t // T_TILE,)
    idx3, val3 = pl.pallas_call(
        _dist_argmin_body,
        grid=grid,
        in_specs=[
            pl.BlockSpec((T_TILE, C_DIM), lambda i: (i, 0)),
            pl.BlockSpec((N_CODES, C_DIM), lambda i: (0, 0)),
        ],
        out_specs=[
            pl.BlockSpec((1, 1, T_TILE), lambda i: (i, 0, 0)),
            pl.BlockSpec((1, 1, T_TILE), lambda i: (i, 0, 0)),
        ],
        out_shape=[
            jax.ShapeDtypeStruct((grid[0], 1, T_TILE), jnp.int32),
            jax.ShapeDtypeStruct((grid[0], 1, T_TILE), jnp.float32),
        ],
    )(z_flat, embedding_weight)
    return idx3.reshape(-1), val3.reshape(-1)


_SC_CHUNK = 256


def _make_sc_gather(n_tokens):
    info = plsc.get_sparse_core_info()
    nc, ns = info.num_cores, info.num_subcores
    nw = nc * ns
    b_per_w = n_tokens // nw
    n_chunks = b_per_w // _SC_CHUNK
    mesh = plsc.VectorSubcoreMesh(core_axis_name="c", subcore_axis_name="s")

    @functools.partial(
        pl.kernel, mesh=mesh,
        out_type=jax.ShapeDtypeStruct((n_tokens, C_DIM), jnp.float32),
        scratch_types=[
            pltpu.VMEM((_SC_CHUNK,), jnp.int32),
            pltpu.VMEM((_SC_CHUNK, C_DIM), jnp.float32),
            pltpu.SemaphoreType.DMA,
        ],
    )
    def gather_k(table_hbm, idx_hbm, out_hbm, idx_v, rows_v, sem):
        wid = lax.axis_index("s") * nc + lax.axis_index("c")
        base = wid * b_per_w
        for ci in range(n_chunks):
            off = base + ci * _SC_CHUNK
            pltpu.sync_copy(idx_hbm.at[pl.ds(off, _SC_CHUNK)], idx_v)
            pltpu.async_copy(table_hbm.at[idx_v], rows_v, sem).wait()
            pltpu.sync_copy(rows_v, out_hbm.at[pl.ds(off, _SC_CHUNK)])

    return gather_k


def kernel(z, embedding_weight):
    b, c, d_, h, w = z.shape
    z_flat = jnp.transpose(z, (0, 2, 3, 4, 1)).reshape(-1, c)
    min_encoding_indices, min_vals = _dist_argmin(z_flat, embedding_weight)
    z_q_flat = _make_sc_gather(z_flat.shape[0])(
        embedding_weight, min_encoding_indices)
    z_q = jnp.transpose(z_q_flat.reshape(b, d_, h, w, c), (0, 4, 1, 2, 3))
    loss = 1.25 * jnp.sum(min_vals) / jnp.float32(z.size)
    indices = min_encoding_indices.reshape(b, d_, h, w)
    return (z_q, loss, indices)
